# baseline (speedup 1.0000x reference)
import functools
import jax
import jax.numpy as jnp
from jax import lax
from jax.experimental import pallas as pl
from jax.experimental.pallas import tpu as pltpu
from jax.experimental.pallas import tpu_sc as plsc

VOCAB = 1000000
D = 64
DPAD = 128
SCALE = 8.0

_info = plsc.get_sparse_core_info()
NC = _info.num_cores
NS = _info.num_subcores
L = _info.num_lanes
NW = NC * NS

XROWS = 4096
XCOLS = 200
B = XROWS * XCOLS
B_PER_W = B // NW
CH = 128
NCHUNK = B_PER_W // CH
NBUF = 2
NOUTER = NCHUNK // NBUF

_mesh = plsc.VectorSubcoreMesh(core_axis_name="c", subcore_axis_name="s")


@functools.partial(
    pl.kernel,
    mesh=_mesh,
    compiler_params=pltpu.CompilerParams(use_tc_tiling_on_sc=True),
    out_type=jax.ShapeDtypeStruct((B, D), jnp.float32),
    scratch_types=[
        pltpu.VMEM((B_PER_W,), jnp.int32),
        pltpu.VMEM((NBUF, CH), jnp.int32),
        pltpu.VMEM((NBUF, CH, DPAD), jnp.float32),
        pltpu.VMEM((NBUF, CH, D), jnp.float32),
        pltpu.SemaphoreType.DMA((NBUF,)),
        pltpu.SemaphoreType.DMA((NBUF,)),
    ],
)
def _embed_kernel(x_hbm, lut_hbm, out_hbm, idx_v, idx2_v, gbuf, obuf, gsem,
                  psem):
    wid = lax.axis_index("s") * NC + lax.axis_index("c")
    base = wid * B_PER_W
    pltpu.sync_copy(x_hbm.at[pl.ds(base, B_PER_W)], idx_v)

    def fill_idx2(j, b):
        for g in range(CH // L):
            v = idx_v[pl.ds(j * CH + g * L, L)]
            idx2_v[b, pl.ds(g * L, L)] = lax.shift_right_logical(v, 1)

    def gather_copy(b):
        return pltpu.make_async_copy(
            lut_hbm.at[idx2_v.at[b]], gbuf.at[b], gsem.at[b])

    def put_copy(j, b):
        return pltpu.make_async_copy(
            obuf.at[b], out_hbm.at[pl.ds(base + j * CH, CH)], psem.at[b])

    def scale_chunk(j, b):
        def grp(g, c):
            hv = (idx_v[pl.ds(j * CH + g * L, L)] & 1) * D
            for u in range(L):
                ii = g * L + u
                h = hv[u]
                for q in range(D // L):
                    obuf[b, ii, pl.ds(q * L, L)] = (
                        gbuf[b, ii, pl.ds(h + q * L, L)] * SCALE)
            return c
        lax.fori_loop(0, CH // L, grp, 0)

    def step(j, b, first, last):
        gather_copy(b).wait()
        if not first:
            put_copy(j, b).wait()
        scale_chunk(j, b)
        if not last:
            fill_idx2(j + NBUF, b)
            gather_copy(b).start()
        put_copy(j, b).start()

    for b in range(NBUF):
        fill_idx2(b, b)
        gather_copy(b).start()

    for b in range(NBUF):
        step(b, b, True, False)

    def outer(g, c):
        for b in range(NBUF):
            step(g * NBUF + b, b, False, False)
        return c

    lax.fori_loop(1, NOUTER - 1, outer, 0)

    for b in range(NBUF):
        step((NOUTER - 1) * NBUF + b, b, False, True)

    for b in range(NBUF):
        put_copy((NOUTER - 1) * NBUF + b, b).wait()


def kernel(x, lut):
    lut2 = lut.reshape(VOCAB // 2, DPAD)
    out = _embed_kernel(x.astype(jnp.int32).reshape(-1), lut2)
    return out.reshape(XROWS, XCOLS, D)


# speedup vs baseline: 1.2639x; 1.2639x over previous
import functools
import jax
import jax.numpy as jnp
from jax import lax
from jax.experimental import pallas as pl
from jax.experimental.pallas import tpu as pltpu
from jax.experimental.pallas import tpu_sc as plsc

VOCAB = 1000000
D = 64
DPAD = 128
SCALE = 8.0

_info = plsc.get_sparse_core_info()
NC = _info.num_cores
NS = _info.num_subcores
L = _info.num_lanes
NW = NC * NS

XROWS = 4096
XCOLS = 200
B = XROWS * XCOLS
B_PER_W = B // NW
CH = 128
NCHUNK = B_PER_W // CH
NBUF = 3
NIB = 2 * NBUF
NFULL = NCHUNK // NBUF
NREM = NCHUNK % NBUF
RU = 4

_mesh = plsc.VectorSubcoreMesh(core_axis_name="c", subcore_axis_name="s")


@functools.partial(
    pl.kernel,
    mesh=_mesh,
    compiler_params=pltpu.CompilerParams(use_tc_tiling_on_sc=True),
    out_type=jax.ShapeDtypeStruct((B, D), jnp.float32),
    scratch_types=[
        pltpu.VMEM((NIB, CH), jnp.int32),
        pltpu.VMEM((NBUF, CH, DPAD), jnp.float32),
        pltpu.VMEM((NBUF, CH, D), jnp.float32),
        pltpu.SemaphoreType.DMA((NIB,)),
        pltpu.SemaphoreType.DMA((NBUF,)),
        pltpu.SemaphoreType.DMA((NBUF,)),
    ],
)
def _embed_kernel(x_hbm, lut_hbm, out_hbm, idx_v, gbuf, obuf, isem, gsem,
                  psem):
    wid = lax.axis_index("s") * NC + lax.axis_index("c")
    base = wid * B_PER_W

    def idx_copy(j, ib):
        return pltpu.make_async_copy(
            x_hbm.at[pl.ds(base + j * CH, CH)], idx_v.at[ib], isem.at[ib])

    def gather_copy(b, ib):
        return pltpu.make_async_copy(
            lut_hbm.at[idx_v.at[ib]], gbuf.at[b], gsem.at[b])

    def put_copy(j, b):
        return pltpu.make_async_copy(
            obuf.at[b], out_hbm.at[pl.ds(base + j * CH, CH)], psem.at[b])

    def scale_chunk(b):
        def mrow(i, c):
            for u in range(RU):
                ii = i * RU + u
                for q in range(D // L):
                    obuf[b, ii, pl.ds(q * L, L)] = (
                        gbuf[b, ii, pl.ds(q * L, L)] * SCALE)
            return c
        lax.fori_loop(0, CH // RU, mrow, 0)

    def step(j, b, ib, first, last):
        gather_copy(b, ib).wait()
        if not first:
            put_copy(j, b).wait()
        scale_chunk(b)
        if not last:
            jn = j + NBUF
            idx_copy(jn, (ib + NBUF) % NIB).wait()
            gather_copy(b, (ib + NBUF) % NIB).start()
            if not first:
                jf = j + NIB

                @pl.when(jf < NCHUNK)
                def _():
                    idx_copy(jf, ib).start()
        put_copy(j, b).start()

    for j in range(NIB):
        idx_copy(j, j).start()
    for b in range(NBUF):
        idx_copy(b, b).wait()
        gather_copy(b, b).start()

    for b in range(NBUF):
        step(b, b, b, True, False)
        idx_copy(b + NIB, b).start()

    nmid = NFULL - 2
    assert nmid % 2 == 0

    def outer2(g2, c):
        g = 1 + g2 * 2
        for b in range(NBUF):
            step(g * NBUF + b, b, b + NBUF, False, False)
        for b in range(NBUF):
            step((g + 1) * NBUF + b, b, b, False, False)
        return c

    lax.fori_loop(0, nmid // 2, outer2, 0)

    glast = NFULL - 1
    for b in range(NBUF):
        step(glast * NBUF + b, b, b + NBUF, False, b >= NREM)

    for b in range(NREM):
        step(NFULL * NBUF + b, b, b, False, True)

    for b in range(NBUF):
        j = NFULL * NBUF + b if b < NREM else glast * NBUF + b
        put_copy(j, b).wait()


def kernel(x, lut):
    lutp = jnp.pad(lut, ((0, 0), (0, DPAD - D)))
    out = _embed_kernel(x.astype(jnp.int32).reshape(-1), lutp)
    return out.reshape(XROWS, XCOLS, D)
